# baseline (speedup 1.0000x reference)
import jax
import jax.numpy as jnp
from jax import lax
from jax.experimental import pallas as pl
from jax.experimental.pallas import tpu as pltpu
from jax.experimental.pallas import tpu_sc as plsc

N_NODES = 10000
N_EDGES = 320000
D = 128
DH = D // 2

_NC = 2
_NS = 16
_EPW = N_EDGES // _NS
_K = 80
_G = 5
_SK = _G * _K
_SCHUNKS = _EPW // _SK
_NSLOT = 2
_RCHUNKS = N_NODES // _K


def _sc_body(x_hbm, src_hbm, dst_hbm, out_hbm, isrc_all, idst0, idst1,
             rows0, rows1, acc, sem_id0, sem_id1, sem_g0, sem_g1,
             sem_s0, sem_s1):
    idst = (idst0, idst1)
    rows = (rows0, rows1)
    sem_id = (sem_id0, sem_id1)
    sem_g = (sem_g0, sem_g1)
    sem_s = (sem_s0, sem_s1)
    c = lax.axis_index("c")
    s = lax.axis_index("s")
    zrow = rows0.at[pl.ds(0, _K)]

    zero = jnp.zeros((16,), jnp.float32)

    def _zfill(i, carry):
        for j in range(DH // 16):
            rows0[i, pl.ds(j * 16, 16)] = zero
        return carry

    lax.fori_loop(0, _K, _zfill, 0)
    for k in range((_RCHUNKS + _NS - 1) // _NS):
        cid = k * _NS + s

        @pl.when(cid < _RCHUNKS)
        def _():
            off = pl.multiple_of(cid * _K, 8)
            pltpu.sync_copy(zrow, acc.at[pl.ds(off, _K)])

    plsc.subcore_barrier()

    xh = x_hbm.at[c]
    cbase = s * _SCHUNKS

    pltpu.sync_copy(src_hbm.at[pl.ds(cbase, _SCHUNKS)], isrc_all)

    def _fire(slot, sc):
        pltpu.async_copy(dst_hbm.at[cbase + sc], idst[slot], sem_id[slot])
        pltpu.async_copy(xh.at[isrc_all.at[sc]], rows[slot], sem_g[slot])

    def _wait_and_scatter(slot, sc):
        pltpu.make_async_copy(xh.at[isrc_all.at[sc]], rows[slot],
                              sem_g[slot]).wait()
        pltpu.make_async_copy(dst_hbm.at[cbase + sc], idst[slot],
                              sem_id[slot]).wait()
        pltpu.async_copy(rows[slot], acc.at[idst[slot]], sem_s[slot], add=True)

    def _wait_scatter(slot):
        pltpu.make_async_copy(rows[slot], acc.at[idst[slot]],
                              sem_s[slot]).wait()

    _fire(0, 0)
    _fire(1, 1)

    def _pair(t, carry):
        sc0 = t * _NSLOT
        _wait_and_scatter(0, sc0)
        _wait_and_scatter(1, sc0 + 1)
        _wait_scatter(0)
        _fire(0, sc0 + 2)
        _wait_scatter(1)
        _fire(1, sc0 + 3)
        return carry

    lax.fori_loop(0, _SCHUNKS // _NSLOT - 1, _pair, 0)

    last = _SCHUNKS - _NSLOT
    _wait_and_scatter(0, last)
    _wait_and_scatter(1, last + 1)
    _wait_scatter(0)
    _wait_scatter(1)
    plsc.subcore_barrier()

    for k in range((_RCHUNKS + _NS - 1) // _NS):
        cid = k * _NS + s

        @pl.when(cid < _RCHUNKS)
        def _():
            off = pl.multiple_of(cid * _K, 8)
            pltpu.sync_copy(acc.at[pl.ds(off, _K)], out_hbm.at[c, pl.ds(off, _K)])


def _sc_aggregate(xsplit, src, dst):
    mesh = plsc.VectorSubcoreMesh(core_axis_name="c", subcore_axis_name="s")
    return pl.kernel(
        _sc_body,
        out_type=jax.ShapeDtypeStruct((_NC, N_NODES, DH), jnp.float32),
        mesh=mesh,
        scratch_types=(
            [pltpu.VMEM((_SCHUNKS, _SK), jnp.int32)]
            + [pltpu.VMEM((_SK,), jnp.int32) for _ in range(_NSLOT)]
            + [pltpu.VMEM((_SK, DH), jnp.float32) for _ in range(_NSLOT)]
            + [pltpu.VMEM_SHARED((N_NODES, DH), jnp.float32)]
            + [pltpu.SemaphoreType.DMA for _ in range(3 * _NSLOT)]
        ),
        compiler_params=pltpu.CompilerParams(use_tc_tiling_on_sc=False),
    )(xsplit, src, dst)


def _tc_body(x_ref, pa_ref, pb_ref, w1_ref, w2_ref, b_ref, o_ref):
    ng = jnp.concatenate([pa_ref[...], pb_ref[...]], axis=1)
    o_ref[...] = (
        jnp.dot(x_ref[...], w1_ref[...], preferred_element_type=jnp.float32)
        + jnp.dot(ng, w2_ref[...], preferred_element_type=jnp.float32)
        + b_ref[...]
    )


def _tc_linear(x, pa, pb, w1t, w2t, b2d):
    blk = 1000
    grid = (N_NODES // blk,)
    return pl.pallas_call(
        _tc_body,
        grid=grid,
        in_specs=[
            pl.BlockSpec((blk, D), lambda i: (i, 0)),
            pl.BlockSpec((blk, DH), lambda i: (i, 0)),
            pl.BlockSpec((blk, DH), lambda i: (i, 0)),
            pl.BlockSpec((D, D), lambda i: (0, 0)),
            pl.BlockSpec((D, D), lambda i: (0, 0)),
            pl.BlockSpec((1, D), lambda i: (0, 0)),
        ],
        out_specs=pl.BlockSpec((blk, D), lambda i: (i, 0)),
        out_shape=jax.ShapeDtypeStruct((N_NODES, D), jnp.float32),
    )(x, pa, pb, w1t, w2t, b2d)


@jax.jit
def kernel(x, edge_index, W, b):
    src = edge_index[0].astype(jnp.int32).reshape(_NS * _SCHUNKS, _SK)
    dst = edge_index[1].astype(jnp.int32).reshape(_NS * _SCHUNKS, _SK)
    xsplit = jnp.stack([x[:, :DH], x[:, DH:]])
    ng_halves = _sc_aggregate(xsplit, src, dst)
    w1t = W[:, :D].T
    w2t = W[:, D:].T
    return _tc_linear(x, ng_halves[0], ng_halves[1], w1t, w2t, b.reshape(1, D))


# speedup vs baseline: 1.1567x; 1.1567x over previous
import jax
import jax.numpy as jnp
from jax import lax
from jax.experimental import pallas as pl
from jax.experimental.pallas import tpu as pltpu
from jax.experimental.pallas import tpu_sc as plsc

N_NODES = 10000
N_EDGES = 320000
D = 128
DH = D // 2

_NC = 2
_NS = 16
_EPW = N_EDGES // _NS
_K = 80
_SK = 200
_SCHUNKS = _EPW // _SK
_NB = 2
_NSLOT = 2 * _NB
_PAIRS = _SCHUNKS // _NSLOT - 1
_RCHUNKS = N_NODES // _K


def _sc_body(x_hbm, src_hbm, dst_hbm, out_hbm, *refs):
    isrc_all = refs[0]
    idst = refs[1:1 + _NSLOT]
    rows = refs[1 + _NSLOT:1 + 2 * _NSLOT]
    acc = refs[1 + 2 * _NSLOT]
    sem_id = refs[2 + 2 * _NSLOT:2 + 3 * _NSLOT]
    sem_g = refs[2 + 3 * _NSLOT:2 + 4 * _NSLOT]
    sem_s = refs[2 + 4 * _NSLOT:2 + 5 * _NSLOT]
    c = lax.axis_index("c")
    s = lax.axis_index("s")
    rows0 = rows[0]
    zrow = rows0.at[pl.ds(0, _K)]

    zero = jnp.zeros((16,), jnp.float32)

    def _zfill(i, carry):
        for j in range(DH // 16):
            rows0[i, pl.ds(j * 16, 16)] = zero
        return carry

    lax.fori_loop(0, _K, _zfill, 0)
    for k in range((_RCHUNKS + _NS - 1) // _NS):
        cid = k * _NS + s

        @pl.when(cid < _RCHUNKS)
        def _():
            off = pl.multiple_of(cid * _K, 8)
            pltpu.sync_copy(zrow, acc.at[pl.ds(off, _K)])

    plsc.subcore_barrier()

    xh = x_hbm.at[c]
    cbase = s * _SCHUNKS

    pltpu.sync_copy(src_hbm.at[pl.ds(cbase, _SCHUNKS)], isrc_all)

    def _fire(slot, sc):
        pltpu.async_copy(dst_hbm.at[cbase + sc], idst[slot], sem_id[slot])
        pltpu.async_copy(xh.at[isrc_all.at[sc]], rows[slot], sem_g[slot])

    def _wait_and_scatter(slot, sc):
        pltpu.make_async_copy(xh.at[isrc_all.at[sc]], rows[slot],
                              sem_g[slot]).wait()
        pltpu.make_async_copy(dst_hbm.at[cbase + sc], idst[slot],
                              sem_id[slot]).wait()
        pltpu.async_copy(rows[slot], acc.at[idst[slot]], sem_s[slot], add=True)

    def _wait_scatter(slot):
        pltpu.make_async_copy(rows[slot], acc.at[idst[slot]],
                              sem_s[slot]).wait()

    for b in range(_NSLOT):
        _fire(b, b)

    def _pair(t, carry):
        base = t * _NSLOT
        nxt = base + _NSLOT
        for b in range(_NB):
            _wait_and_scatter(b, base + b)
        for b in range(_NB, _NSLOT):
            _wait_and_scatter(b, base + b)
        for b in range(_NB):
            _wait_scatter(b)
            _fire(b, nxt + b)
        for b in range(_NB, _NSLOT):
            _wait_scatter(b)
            _fire(b, nxt + b)
        return carry

    lax.fori_loop(0, _PAIRS, _pair, 0)

    last = _PAIRS * _NSLOT
    for b in range(_NSLOT):
        _wait_and_scatter(b, last + b)
    for b in range(_NSLOT):
        _wait_scatter(b)
    plsc.subcore_barrier()

    for k in range((_RCHUNKS + _NS - 1) // _NS):
        cid = k * _NS + s

        @pl.when(cid < _RCHUNKS)
        def _():
            off = pl.multiple_of(cid * _K, 8)
            pltpu.sync_copy(acc.at[pl.ds(off, _K)], out_hbm.at[c, pl.ds(off, _K)])


def _sc_aggregate(xsplit, src, dst):
    mesh = plsc.VectorSubcoreMesh(core_axis_name="c", subcore_axis_name="s")
    return pl.kernel(
        _sc_body,
        out_type=jax.ShapeDtypeStruct((_NC, N_NODES, DH), jnp.float32),
        mesh=mesh,
        scratch_types=(
            [pltpu.VMEM((_SCHUNKS, _SK), jnp.int32)]
            + [pltpu.VMEM((_SK,), jnp.int32) for _ in range(_NSLOT)]
            + [pltpu.VMEM((_SK, DH), jnp.float32) for _ in range(_NSLOT)]
            + [pltpu.VMEM_SHARED((N_NODES, DH), jnp.float32)]
            + [pltpu.SemaphoreType.DMA for _ in range(3 * _NSLOT)]
        ),
        compiler_params=pltpu.CompilerParams(use_tc_tiling_on_sc=False),
    )(xsplit, src, dst)


def _tc_body(x_ref, pa_ref, pb_ref, w1_ref, w2_ref, b_ref, o_ref):
    ng = jnp.concatenate([pa_ref[...], pb_ref[...]], axis=1)
    o_ref[...] = (
        jnp.dot(x_ref[...], w1_ref[...], preferred_element_type=jnp.float32)
        + jnp.dot(ng, w2_ref[...], preferred_element_type=jnp.float32)
        + b_ref[...]
    )


def _tc_linear(x, pa, pb, w1t, w2t, b2d):
    blk = 1000
    grid = (N_NODES // blk,)
    return pl.pallas_call(
        _tc_body,
        grid=grid,
        in_specs=[
            pl.BlockSpec((blk, D), lambda i: (i, 0)),
            pl.BlockSpec((blk, DH), lambda i: (i, 0)),
            pl.BlockSpec((blk, DH), lambda i: (i, 0)),
            pl.BlockSpec((D, D), lambda i: (0, 0)),
            pl.BlockSpec((D, D), lambda i: (0, 0)),
            pl.BlockSpec((1, D), lambda i: (0, 0)),
        ],
        out_specs=pl.BlockSpec((blk, D), lambda i: (i, 0)),
        out_shape=jax.ShapeDtypeStruct((N_NODES, D), jnp.float32),
    )(x, pa, pb, w1t, w2t, b2d)


@jax.jit
def kernel(x, edge_index, W, b):
    src = edge_index[0].astype(jnp.int32).reshape(_NS * _SCHUNKS, _SK)
    dst = edge_index[1].astype(jnp.int32).reshape(_NS * _SCHUNKS, _SK)
    xsplit = jnp.stack([x[:, :DH], x[:, DH:]])
    ng_halves = _sc_aggregate(xsplit, src, dst)
    w1t = W[:, :D].T
    w2t = W[:, D:].T
    return _tc_linear(x, ng_halves[0], ng_halves[1], w1t, w2t, b.reshape(1, D))


# speedup vs baseline: 1.2758x; 1.1029x over previous
import jax
import jax.numpy as jnp
from jax import lax
from jax.experimental import pallas as pl
from jax.experimental.pallas import tpu as pltpu
from jax.experimental.pallas import tpu_sc as plsc

N_NODES = 10000
N_EDGES = 320000
D = 128

_NC = 2
_NS = 16
_NW = _NC * _NS
_EPW = N_EDGES // _NW
_K = 80
_CHUNKS = _EPW // _K
_NB = 2
_NSLOT = 2 * _NB
_PAIRS = _CHUNKS // _NSLOT
_RCHUNKS = N_NODES // _K


def _sc_body(x_hbm, src_hbm, dst_hbm, out_hbm, *refs):
    isrc = refs[:_NSLOT]
    idst = refs[_NSLOT:2 * _NSLOT]
    rows = refs[2 * _NSLOT:3 * _NSLOT]
    acc = refs[3 * _NSLOT]
    sem_i = refs[1 + 3 * _NSLOT:1 + 4 * _NSLOT]
    sem_g = refs[1 + 4 * _NSLOT:1 + 5 * _NSLOT]
    sem_s = refs[1 + 5 * _NSLOT:1 + 6 * _NSLOT]
    c = lax.axis_index("c")
    s = lax.axis_index("s")
    rows0 = rows[0]

    zero = jnp.zeros((16,), jnp.float32)

    def _zfill(i, carry):
        for j in range(D // 16):
            rows0[i, pl.ds(j * 16, 16)] = zero
        return carry

    lax.fori_loop(0, _K, _zfill, 0)
    for k in range((_RCHUNKS + _NS - 1) // _NS):
        cid = k * _NS + s

        @pl.when(cid < _RCHUNKS)
        def _():
            off = pl.multiple_of(cid * _K, 8)
            pltpu.sync_copy(rows0, acc.at[pl.ds(off, _K)])

    plsc.subcore_barrier()

    ebase = (c * _NS + s) * _EPW

    def _fire_idx(slot, chunk):
        off = ebase + chunk * _K
        pltpu.async_copy(src_hbm.at[pl.ds(off, _K)], isrc[slot], sem_i[slot])
        pltpu.async_copy(dst_hbm.at[pl.ds(off, _K)], idst[slot], sem_i[slot])

    def _fire_gather(slot, chunk):
        off = ebase + chunk * _K
        pltpu.make_async_copy(src_hbm.at[pl.ds(off, _K)], isrc[slot],
                              sem_i[slot]).wait()
        pltpu.async_copy(x_hbm.at[isrc[slot]], rows[slot], sem_g[slot])

    def _wait_and_scatter(slot, chunk):
        off = ebase + chunk * _K
        pltpu.make_async_copy(x_hbm.at[isrc[slot]], rows[slot],
                              sem_g[slot]).wait()
        pltpu.make_async_copy(dst_hbm.at[pl.ds(off, _K)], idst[slot],
                              sem_i[slot]).wait()
        pltpu.async_copy(rows[slot], acc.at[idst[slot]], sem_s[slot], add=True)

    def _wait_scatter(slot):
        pltpu.make_async_copy(rows[slot], acc.at[idst[slot]],
                              sem_s[slot]).wait()

    for b in range(_NSLOT):
        _fire_idx(b, b)
    for b in range(_NSLOT):
        _fire_gather(b, b)

    def _round(t, carry):
        base = t * _NSLOT
        nxt = base + _NSLOT
        for b in range(_NB):
            _wait_and_scatter(b, base + b)
        for b in range(_NB, _NSLOT):
            _wait_and_scatter(b, base + b)
        for b in range(_NB):
            _wait_scatter(b)
            _fire_idx(b, nxt + b)
        for b in range(_NB):
            _fire_gather(b, nxt + b)
        for b in range(_NB, _NSLOT):
            _wait_scatter(b)
            _fire_idx(b, nxt + b)
        for b in range(_NB, _NSLOT):
            _fire_gather(b, nxt + b)
        return carry

    lax.fori_loop(0, _PAIRS - 1, _round, 0)

    last = (_PAIRS - 1) * _NSLOT
    for b in range(_NSLOT):
        _wait_and_scatter(b, last + b)
    _wait_scatter(0)
    _fire_idx(0, last + _NSLOT)
    _fire_gather(0, last + _NSLOT)
    _wait_and_scatter(0, last + _NSLOT)
    for b in range(1, _NSLOT):
        _wait_scatter(b)
    _wait_scatter(0)
    plsc.subcore_barrier()

    for k in range((_RCHUNKS + _NS - 1) // _NS):
        cid = k * _NS + s

        @pl.when(cid < _RCHUNKS)
        def _():
            off = pl.multiple_of(cid * _K, 8)
            pltpu.sync_copy(acc.at[pl.ds(off, _K)], out_hbm.at[c, pl.ds(off, _K)])


def _sc_aggregate(x, src, dst):
    mesh = plsc.VectorSubcoreMesh(core_axis_name="c", subcore_axis_name="s")
    return pl.kernel(
        _sc_body,
        out_type=jax.ShapeDtypeStruct((_NC, N_NODES, D), jnp.float32),
        mesh=mesh,
        scratch_types=(
            [pltpu.VMEM((_K,), jnp.int32) for _ in range(2 * _NSLOT)]
            + [pltpu.VMEM((_K, D), jnp.float32) for _ in range(_NSLOT)]
            + [pltpu.VMEM_SHARED((N_NODES, D), jnp.float32)]
            + [pltpu.SemaphoreType.DMA for _ in range(3 * _NSLOT)]
        ),
    )(x, src, dst)


def _tc_body(x_ref, p_ref, w1_ref, w2_ref, b_ref, o_ref):
    ng = p_ref[0] + p_ref[1]
    o_ref[...] = (
        jnp.dot(x_ref[...], w1_ref[...], preferred_element_type=jnp.float32)
        + jnp.dot(ng, w2_ref[...], preferred_element_type=jnp.float32)
        + b_ref[...]
    )


def _tc_linear(x, partials, w1t, w2t, b2d):
    blk = 1000
    grid = (N_NODES // blk,)
    return pl.pallas_call(
        _tc_body,
        grid=grid,
        in_specs=[
            pl.BlockSpec((blk, D), lambda i: (i, 0)),
            pl.BlockSpec((_NC, blk, D), lambda i: (0, i, 0)),
            pl.BlockSpec((D, D), lambda i: (0, 0)),
            pl.BlockSpec((D, D), lambda i: (0, 0)),
            pl.BlockSpec((1, D), lambda i: (0, 0)),
        ],
        out_specs=pl.BlockSpec((blk, D), lambda i: (i, 0)),
        out_shape=jax.ShapeDtypeStruct((N_NODES, D), jnp.float32),
    )(x, partials, w1t, w2t, b2d)


@jax.jit
def kernel(x, edge_index, W, b):
    src = edge_index[0].astype(jnp.int32)
    dst = edge_index[1].astype(jnp.int32)
    partials = _sc_aggregate(x, src, dst)
    w1t = W[:, :D].T
    w2t = W[:, D:].T
    return _tc_linear(x, partials, w1t, w2t, b.reshape(1, D))
